# baseline (speedup 1.0000x reference)
import functools

import jax
import jax.numpy as jnp
from jax.experimental import pallas as pl

EPS = 1e-09

_C = (
    8.121172e-07,
    1.4426336,
    -0.72020257,
    0.47172153,
    -0.32148296,
    0.18865228,
    -0.07592081,
    0.014598641,
)


def _fast_log2(x):
    xb = jax.lax.bitcast_convert_type(x, jnp.int32)
    e = jax.lax.shift_right_arithmetic(xb, 23) - 127
    mb = jax.lax.bitwise_or(
        jax.lax.bitwise_and(xb, 0x007FFFFF), 0x3F800000
    )
    m = jax.lax.bitcast_convert_type(mb, jnp.float32)
    t = m - 1.0
    p = jnp.float32(_C[7])
    for c in _C[6::-1]:
        p = p * t + jnp.float32(c)
    return e.astype(jnp.float32) + p


def _logmse_block(y_pred_ref, y_true_ref, out_ref, *, nblocks, inv_n):
    i = pl.program_id(0)

    yp = y_pred_ref[...]
    yt = y_true_ref[...]
    lp = _fast_log2(jnp.maximum(yp, EPS))
    lt = _fast_log2(jnp.maximum(yt, EPS))
    d = lt - lp
    partial = jnp.sum(d * d).reshape(1, 1)

    @pl.when(i == 0)
    def _init():
        out_ref[...] = partial

    @pl.when(i > 0)
    def _acc():
        out_ref[...] = out_ref[...] + partial

    @pl.when(i == nblocks - 1)
    def _finish():
        out_ref[...] = out_ref[...] * inv_n


def kernel(y_pred, y_true):
    b, s, _ = y_pred.shape
    lanes = 128
    reps = s // lanes
    n = b * reps
    yp = y_pred.reshape(n, lanes)
    yt = jnp.broadcast_to(y_true.reshape(b, 1, 1), (b, reps, 1)).reshape(n, 1)
    rows = 4096
    nblocks = n // rows
    inv_n = 1.0 / float(b * s)
    out = pl.pallas_call(
        functools.partial(_logmse_block, nblocks=nblocks, inv_n=inv_n),
        grid=(nblocks,),
        in_specs=[
            pl.BlockSpec((rows, lanes), lambda i: (i, 0)),
            pl.BlockSpec((rows, 1), lambda i: (i, 0)),
        ],
        out_specs=pl.BlockSpec((1, 1), lambda i: (0, 0)),
        out_shape=jax.ShapeDtypeStruct((1, 1), jnp.float32),
    )(yp, yt)
    loss = out[0, 0]
    return (loss, loss, jnp.array(0.0, dtype=jnp.float32))


# speedup vs baseline: 1.4170x; 1.4170x over previous
import functools

import jax
import jax.numpy as jnp
from jax.experimental import pallas as pl

EPS = 1e-09


def _logmse_block(y_pred_ref, y_true_ref, out_ref, *, nblocks, inv_n):
    i = pl.program_id(0)

    yp = y_pred_ref[...]
    yt = y_true_ref[...]
    lp = jnp.log2(jnp.maximum(yp, EPS))
    lt = jnp.log2(jnp.maximum(yt, EPS))
    d = lt - lp
    partial = jnp.sum(d * d).reshape(1, 1)

    @pl.when(i == 0)
    def _init():
        out_ref[...] = partial

    @pl.when(i > 0)
    def _acc():
        out_ref[...] = out_ref[...] + partial

    @pl.when(i == nblocks - 1)
    def _finish():
        out_ref[...] = out_ref[...] * inv_n


def kernel(y_pred, y_true):
    b, s, _ = y_pred.shape
    lanes = 128
    reps = s // lanes
    n = b * reps
    yp = y_pred.reshape(n, lanes)
    yt = jnp.broadcast_to(y_true.reshape(b, 1, 1), (b, reps, 1)).reshape(n, 1)
    rows = 8192
    nblocks = n // rows
    inv_n = 1.0 / float(b * s)
    out = pl.pallas_call(
        functools.partial(_logmse_block, nblocks=nblocks, inv_n=inv_n),
        grid=(nblocks,),
        in_specs=[
            pl.BlockSpec((rows, lanes), lambda i: (i, 0)),
            pl.BlockSpec((rows, 1), lambda i: (i, 0)),
        ],
        out_specs=pl.BlockSpec((1, 1), lambda i: (0, 0)),
        out_shape=jax.ShapeDtypeStruct((1, 1), jnp.float32),
    )(yp, yt)
    loss = out[0, 0]
    return (loss, loss, jnp.array(0.0, dtype=jnp.float32))
